# baseline (speedup 1.0000x reference)
import jax
import jax.numpy as jnp
from jax.experimental import pallas as pl
from jax.experimental.pallas import tpu as pltpu

_CB, _DB = 2, 256
_IN = _CB + _DB
_ST = 256
_H, _M, _SH = 4, 32, 3
_READ = _H * _M
_COMB = _IN + _ST + _READ
_UPH = 106
_EPS = 1e-12
_ZW = 1024

_S_OFF = 512
_JD_OFF = 524
_J_OFF = 528
_GA_OFF = 540
_BE_OFF = 544
_G_OFF = 548
_E_OFF = 640
_AD_OFF = 768
_K_OFF = 896


def _pack_weights(Ws, Wo, Wu, bs, bo, bu):
    Wu_r = Wu.reshape(_COMB, _H, _UPH)
    bu_r = bu.reshape(_H, _UPH)
    zpad = jnp.zeros((_COMB, 88), jnp.float32)

    def grab(lo, hi):
        return Wu_r[:, :, lo:hi].reshape(_COMB, _H * (hi - lo))

    W_all = jnp.concatenate(
        [
            Ws, Wo,
            grab(0, 3),
            Wu_r[:, :, 3],
            grab(4, 7),
            Wu_r[:, :, 7],
            Wu_r[:, :, 104],
            Wu_r[:, :, 105],
            zpad,
            grab(8, 40),
            grab(40, 72),
            grab(72, 104),
        ],
        axis=1,
    )
    b_all = jnp.concatenate(
        [
            bs, bo,
            bu_r[:, 0:3].reshape(-1), bu_r[:, 3], bu_r[:, 4:7].reshape(-1),
            bu_r[:, 7], bu_r[:, 104], bu_r[:, 105], jnp.zeros((88,), jnp.float32),
            bu_r[:, 8:40].reshape(-1), bu_r[:, 40:72].reshape(-1),
            bu_r[:, 72:104].reshape(-1),
        ]
    ).reshape(1, _ZW)
    return W_all, b_all


def _dwm_step_kernel(x_ref, wx_ref, wst_ref, wr_ref, b_ref, out_ref,
                     state_ref, wt_ref, wtd_ref, mem_ref):
    t = pl.program_id(1)
    bh = state_ref.shape[0]
    a = wt_ref.shape[2]

    @pl.when(t == 0)
    def _init():
        state_ref[...] = jnp.ones_like(state_ref)
        lane = jax.lax.broadcasted_iota(jnp.int32, (bh, _H, a), 2)
        w0 = jnp.where(lane == 0, 1.0, 0.0).astype(jnp.float32)
        wt_ref[...] = w0
        wtd_ref[...] = w0
        mem_ref[...] = jnp.full_like(mem_ref, 0.01)

    state = state_ref[...]
    wt = wt_ref[...]
    wt_d = wtd_ref[...]
    mem = mem_ref[...]

    hi = jax.lax.Precision.HIGHEST

    read = jax.lax.dot_general(
        wt, mem, (((2,), (2,)), ((0,), (0,))),
        precision=hi, preferred_element_type=jnp.float32)
    read_flat = read.reshape(bh, _READ)

    x = x_ref[0]
    z = (jnp.dot(x, wx_ref[...], precision=hi, preferred_element_type=jnp.float32)
         + jnp.dot(state, wst_ref[...], precision=hi, preferred_element_type=jnp.float32)
         + jnp.dot(read_flat, wr_ref[...], precision=hi, preferred_element_type=jnp.float32)
         + b_ref[...])

    new_state = jax.nn.sigmoid(z[:, 0:_ST])
    out = z[:, _ST:_ST + _DB]

    s_ = jax.nn.softmax(jax.nn.softplus(z[:, _S_OFF:_S_OFF + 12].reshape(bh, _H, _SH)), axis=-1)
    jd = jax.nn.sigmoid(z[:, _JD_OFF:_JD_OFF + _H])[:, :, None]
    j = jax.nn.softmax(z[:, _J_OFF:_J_OFF + 12].reshape(bh, _H, _SH), axis=-1)
    gamma = (1.0 + jax.nn.softplus(z[:, _GA_OFF:_GA_OFF + _H]))[:, :, None]
    beta = jax.nn.softplus(z[:, _BE_OFF:_BE_OFF + _H])[:, :, None]
    g = jax.nn.sigmoid(z[:, _G_OFF:_G_OFF + _H])[:, :, None]
    erase = jax.nn.sigmoid(z[:, _E_OFF:_E_OFF + _READ]).reshape(bh, _H, _M)
    add = z[:, _AD_OFF:_AD_OFF + _READ].reshape(bh, _H, _M)
    k = jnp.tanh(z[:, _K_OFF:_K_OFF + _READ]).reshape(bh, _H, _M)

    wt_d_new = (1.0 - jd) * wt_d + jd * wt
    lane = jax.lax.broadcasted_iota(jnp.int32, (bh, _H, a), 2)
    wt_addr0 = jnp.where(lane == 0, 1.0, 0.0).astype(jnp.float32)
    wt_j = j[..., 0:1] * wt + j[..., 1:2] * wt_d_new + j[..., 2:3] * wt_addr0

    erase_t = jnp.swapaxes(erase, 1, 2)
    subsets = [(), (0,), (1,), (2,), (3,), (0, 1), (0, 2), (0, 3), (1, 2),
               (1, 3), (2, 3), (0, 1, 2), (0, 1, 3), (0, 2, 3), (1, 2, 3),
               (0, 1, 2, 3)]
    ecols = []
    wrows = []
    for S in subsets:
        if not S:
            ec = jnp.ones((bh, _M, 1), jnp.float32)
            wr = jnp.ones((bh, 1, a), jnp.float32)
        else:
            ec = erase_t[:, :, S[0]:S[0] + 1]
            wr = wt_j[:, S[0]:S[0] + 1, :]
            for h in S[1:]:
                ec = ec * erase_t[:, :, h:h + 1]
                wr = wr * wt_j[:, h:h + 1, :]
            if len(S) % 2 == 1:
                ec = -ec
        ecols.append(ec)
        wrows.append(wr)
    E = jnp.concatenate(ecols, axis=2)
    W = jnp.concatenate(wrows, axis=1)
    keep = jax.lax.dot_general(
        E, W, (((2,), (1,)), ((0,), (0,))),
        precision=hi, preferred_element_type=jnp.float32)
    add_t = jnp.swapaxes(add, 1, 2)
    addterm = jax.lax.dot_general(
        add_t, wt_j, (((2,), (1,)), ((0,), (0,))),
        precision=hi, preferred_element_type=jnp.float32)
    mem_new = mem * keep + addterm

    kn = k / (jnp.sqrt(jnp.sum(k * k, axis=-1, keepdims=True)) + _EPS)
    mn = mem_new / (jnp.sqrt(jnp.sum(mem_new * mem_new, axis=1, keepdims=True)) + _EPS)
    wt_k = jax.lax.dot_general(
        kn, mn, (((2,), (1,)), ((0,), (0,))),
        precision=hi, preferred_element_type=jnp.float32)
    wt_b = jax.nn.softmax(beta * wt_k, axis=-1)
    wt_c = g * wt_b + (1.0 - g) * wt_j

    left = jnp.concatenate([wt_c[..., 1:], wt_c[..., :1]], axis=-1)
    right = jnp.concatenate([wt_c[..., -1:], wt_c[..., :-1]], axis=-1)
    wt_s = s_[..., 0:1] * left + s_[..., 1:2] * wt_c + s_[..., 2:3] * right
    wt_sh = jnp.exp(gamma * jnp.log(wt_s + _EPS))
    wt_new = wt_sh / jnp.sum(wt_sh, axis=-1, keepdims=True)

    state_ref[...] = new_state
    wt_ref[...] = wt_new
    wtd_ref[...] = wt_d_new
    mem_ref[...] = mem_new
    out_ref[0] = out


def kernel(inputs, targets, Ws, bs, Wo, bo, Wu, bu):
    del targets
    B, T, _ = inputs.shape
    BH = B // 2
    A = T

    W_all, b_all = _pack_weights(Ws, Wo, Wu, bs, bo, bu)
    Wx = W_all[0:_IN]
    Wst = W_all[_IN:_IN + _ST]
    Wr = W_all[_IN + _ST:_COMB]

    xs = jnp.swapaxes(inputs, 0, 1)

    outs = pl.pallas_call(
        _dwm_step_kernel,
        grid=(2, T),
        in_specs=[
            pl.BlockSpec((1, BH, _IN), lambda i, t: (t, i, 0)),
            pl.BlockSpec((_IN, _ZW), lambda i, t: (0, 0)),
            pl.BlockSpec((_ST, _ZW), lambda i, t: (0, 0)),
            pl.BlockSpec((_READ, _ZW), lambda i, t: (0, 0)),
            pl.BlockSpec((1, _ZW), lambda i, t: (0, 0)),
        ],
        out_specs=pl.BlockSpec((1, BH, _DB), lambda i, t: (t, i, 0)),
        out_shape=jax.ShapeDtypeStruct((T, B, _DB), jnp.float32),
        scratch_shapes=[
            pltpu.VMEM((BH, _ST), jnp.float32),
            pltpu.VMEM((BH, _H, A), jnp.float32),
            pltpu.VMEM((BH, _H, A), jnp.float32),
            pltpu.VMEM((BH, _M, A), jnp.float32),
        ],
        compiler_params=pltpu.CompilerParams(
            dimension_semantics=("parallel", "arbitrary"),
        ),
    )(xs, Wx, Wst, Wr, b_all)

    return jnp.swapaxes(outs, 0, 1)


# speedup vs baseline: 2.3032x; 2.3032x over previous
import jax
import jax.numpy as jnp
import jax.experimental.shard_map
from jax.experimental import pallas as pl
from jax.experimental.pallas import tpu as pltpu

_CB, _DB = 2, 256
_IN = _CB + _DB
_ST = 256
_H, _M, _SH = 4, 32, 3
_READ = _H * _M
_COMB = _IN + _ST + _READ
_UPH = 106
_EPS = 1e-12
_ZW = 1024

_S_OFF = 512
_JD_OFF = 524
_J_OFF = 528
_GA_OFF = 540
_BE_OFF = 544
_G_OFF = 548
_E_OFF = 640
_AD_OFF = 768
_K_OFF = 896


def _pack_weights(Ws, Wo, Wu, bs, bo, bu):
    Wu_r = Wu.reshape(_COMB, _H, _UPH)
    bu_r = bu.reshape(_H, _UPH)
    zpad = jnp.zeros((_COMB, 88), jnp.float32)

    def grab(lo, hi):
        return Wu_r[:, :, lo:hi].reshape(_COMB, _H * (hi - lo))

    W_all = jnp.concatenate(
        [
            Ws, Wo,
            grab(0, 3),
            Wu_r[:, :, 3],
            grab(4, 7),
            Wu_r[:, :, 7],
            Wu_r[:, :, 104],
            Wu_r[:, :, 105],
            zpad,
            grab(8, 40),
            grab(40, 72),
            grab(72, 104),
        ],
        axis=1,
    )
    b_all = jnp.concatenate(
        [
            bs, bo,
            bu_r[:, 0:3].reshape(-1), bu_r[:, 3], bu_r[:, 4:7].reshape(-1),
            bu_r[:, 7], bu_r[:, 104], bu_r[:, 105], jnp.zeros((88,), jnp.float32),
            bu_r[:, 8:40].reshape(-1), bu_r[:, 40:72].reshape(-1),
            bu_r[:, 72:104].reshape(-1),
        ]
    ).reshape(1, _ZW)
    return W_all, b_all


def _dwm_step_kernel(x_ref, wx_ref, wst_ref, wr_ref, b_ref, out_ref,
                     state_ref, wt_ref, wtd_ref, mem_ref):
    t = pl.program_id(0)
    bh = state_ref.shape[0]
    a = wt_ref.shape[2]

    @pl.when(t == 0)
    def _init():
        state_ref[...] = jnp.ones_like(state_ref)
        lane = jax.lax.broadcasted_iota(jnp.int32, (bh, _H, a), 2)
        w0 = jnp.where(lane == 0, 1.0, 0.0).astype(jnp.float32)
        wt_ref[...] = w0
        wtd_ref[...] = w0
        mem_ref[...] = jnp.full_like(mem_ref, 0.01)

    state = state_ref[...]
    wt = wt_ref[...]
    wt_d = wtd_ref[...]
    mem = mem_ref[...]

    hi = jax.lax.Precision.HIGHEST

    read = jnp.sum(wt[:, :, None, :] * mem[:, None, :, :], axis=-1)
    read_flat = read.reshape(bh, _READ)

    x = x_ref[0]
    z = (jnp.dot(x, wx_ref[...], precision=hi, preferred_element_type=jnp.float32)
         + jnp.dot(state, wst_ref[...], precision=hi, preferred_element_type=jnp.float32)
         + jnp.dot(read_flat, wr_ref[...], precision=hi, preferred_element_type=jnp.float32)
         + b_ref[...])

    new_state = jax.nn.sigmoid(z[:, 0:_ST])
    out = z[:, _ST:_ST + _DB]

    s_ = jax.nn.softmax(jax.nn.softplus(z[:, _S_OFF:_S_OFF + 12].reshape(bh, _H, _SH)), axis=-1)
    jd = jax.nn.sigmoid(z[:, _JD_OFF:_JD_OFF + _H])[:, :, None]
    j = jax.nn.softmax(z[:, _J_OFF:_J_OFF + 12].reshape(bh, _H, _SH), axis=-1)
    gamma = (1.0 + jax.nn.softplus(z[:, _GA_OFF:_GA_OFF + _H]))[:, :, None]
    beta = jax.nn.softplus(z[:, _BE_OFF:_BE_OFF + _H])[:, :, None]
    g = jax.nn.sigmoid(z[:, _G_OFF:_G_OFF + _H])[:, :, None]
    erase = jax.nn.sigmoid(z[:, _E_OFF:_E_OFF + _READ]).reshape(bh, _H, _M)
    add = z[:, _AD_OFF:_AD_OFF + _READ].reshape(bh, _H, _M)
    k = jnp.tanh(z[:, _K_OFF:_K_OFF + _READ]).reshape(bh, _H, _M)

    wt_d_new = (1.0 - jd) * wt_d + jd * wt
    lane = jax.lax.broadcasted_iota(jnp.int32, (bh, _H, a), 2)
    wt_addr0 = jnp.where(lane == 0, 1.0, 0.0).astype(jnp.float32)
    wt_j = j[..., 0:1] * wt + j[..., 1:2] * wt_d_new + j[..., 2:3] * wt_addr0

    erase_t = jnp.swapaxes(erase, 1, 2)
    add_t = jnp.swapaxes(add, 1, 2)
    keep = 1.0
    addterm = 0.0
    for h in range(_H):
        wj_h = wt_j[:, h:h + 1, :]
        keep = keep * (1.0 - erase_t[:, :, h:h + 1] * wj_h)
        addterm = addterm + add_t[:, :, h:h + 1] * wj_h
    mem_new = mem * keep + addterm

    kn = k / (jnp.sqrt(jnp.sum(k * k, axis=-1, keepdims=True)) + _EPS)
    mn = mem_new / (jnp.sqrt(jnp.sum(mem_new * mem_new, axis=1, keepdims=True)) + _EPS)
    wt_k = jax.lax.dot_general(
        kn, mn, (((2,), (1,)), ((0,), (0,))),
        precision=hi, preferred_element_type=jnp.float32)
    wt_b = jax.nn.softmax(beta * wt_k, axis=-1)
    wt_c = g * wt_b + (1.0 - g) * wt_j

    left = jnp.concatenate([wt_c[..., 1:], wt_c[..., :1]], axis=-1)
    right = jnp.concatenate([wt_c[..., -1:], wt_c[..., :-1]], axis=-1)
    wt_s = s_[..., 0:1] * left + s_[..., 1:2] * wt_c + s_[..., 2:3] * right
    wt_sh = jnp.exp(gamma * jnp.log(wt_s + _EPS))
    wt_new = wt_sh / jnp.sum(wt_sh, axis=-1, keepdims=True)

    state_ref[...] = new_state
    wt_ref[...] = wt_new
    wtd_ref[...] = wt_d_new
    mem_ref[...] = mem_new
    out_ref[0] = out


def _run_scan(xs, Wx, Wst, Wr, b_all):
    T, BH, _ = xs.shape
    A = T
    return pl.pallas_call(
        _dwm_step_kernel,
        grid=(T,),
        in_specs=[
            pl.BlockSpec((1, BH, _IN), lambda t: (t, 0, 0)),
            pl.BlockSpec((_IN, _ZW), lambda t: (0, 0)),
            pl.BlockSpec((_ST, _ZW), lambda t: (0, 0)),
            pl.BlockSpec((_READ, _ZW), lambda t: (0, 0)),
            pl.BlockSpec((1, _ZW), lambda t: (0, 0)),
        ],
        out_specs=pl.BlockSpec((1, BH, _DB), lambda t: (t, 0, 0)),
        out_shape=jax.ShapeDtypeStruct((T, BH, _DB), jnp.float32),
        scratch_shapes=[
            pltpu.VMEM((BH, _ST), jnp.float32),
            pltpu.VMEM((BH, _H, A), jnp.float32),
            pltpu.VMEM((BH, _H, A), jnp.float32),
            pltpu.VMEM((BH, _M, A), jnp.float32),
        ],
        compiler_params=pltpu.CompilerParams(
            dimension_semantics=("arbitrary",),
        ),
    )(xs, Wx, Wst, Wr, b_all)


def kernel(inputs, targets, Ws, bs, Wo, bo, Wu, bu):
    del targets
    B, T, _ = inputs.shape

    W_all, b_all = _pack_weights(Ws, Wo, Wu, bs, bo, bu)
    Wx = W_all[0:_IN]
    Wst = W_all[_IN:_IN + _ST]
    Wr = W_all[_IN + _ST:_COMB]

    xs = jnp.swapaxes(inputs, 0, 1)

    devs = jax.devices()
    nsh = 2 if len(devs) >= 2 and B % 2 == 0 else 1
    mesh = jax.sharding.Mesh(devs[:nsh], ("b",))
    P = jax.sharding.PartitionSpec
    outs = jax.experimental.shard_map.shard_map(
        _run_scan,
        mesh=mesh,
        in_specs=(P(None, "b", None), P(None, None), P(None, None),
                  P(None, None), P(None, None)),
        out_specs=P(None, "b", None),
        check_rep=False,
    )(xs, Wx, Wst, Wr, b_all)

    return jnp.swapaxes(outs, 0, 1)


# speedup vs baseline: 2.8698x; 1.2460x over previous
import jax
import jax.numpy as jnp
import jax.experimental.shard_map
from jax.experimental import pallas as pl
from jax.experimental.pallas import tpu as pltpu

_CB, _DB = 2, 256
_IN = _CB + _DB
_ST = 256
_H, _M, _SH = 4, 32, 3
_READ = _H * _M
_COMB = _IN + _ST + _READ
_UPH = 106
_EPS = 1e-12
_ZW = 1024

_S_OFF = 512
_JD_OFF = 524
_J_OFF = 528
_GA_OFF = 540
_BE_OFF = 544
_G_OFF = 548
_E_OFF = 640
_AD_OFF = 768
_K_OFF = 896


def _pack_weights(Ws, Wo, Wu, bs, bo, bu):
    Wu_r = Wu.reshape(_COMB, _H, _UPH)
    bu_r = bu.reshape(_H, _UPH)
    zpad = jnp.zeros((_COMB, 88), jnp.float32)

    def grab(lo, hi):
        return Wu_r[:, :, lo:hi].reshape(_COMB, _H * (hi - lo))

    W_all = jnp.concatenate(
        [
            Ws, Wo,
            grab(0, 3),
            Wu_r[:, :, 3],
            grab(4, 7),
            Wu_r[:, :, 7],
            Wu_r[:, :, 104],
            Wu_r[:, :, 105],
            zpad,
            grab(8, 40),
            grab(40, 72),
            grab(72, 104),
        ],
        axis=1,
    )
    b_all = jnp.concatenate(
        [
            bs, bo,
            bu_r[:, 0:3].reshape(-1), bu_r[:, 3], bu_r[:, 4:7].reshape(-1),
            bu_r[:, 7], bu_r[:, 104], bu_r[:, 105], jnp.zeros((88,), jnp.float32),
            bu_r[:, 8:40].reshape(-1), bu_r[:, 40:72].reshape(-1),
            bu_r[:, 72:104].reshape(-1),
        ]
    ).reshape(1, _ZW)
    return W_all, b_all


def _dwm_step_kernel(x_ref, wx_ref, wx_lo_ref, wst_ref, wst_lo_ref,
                     wr_ref, wr_lo_ref, b_ref, out_ref,
                     state_ref, wt_ref, wtd_ref, mem_ref):
    t = pl.program_id(0)
    bh = state_ref.shape[0]
    a = wt_ref.shape[2]

    @pl.when(t == 0)
    def _init():
        state_ref[...] = jnp.ones_like(state_ref)
        lane = jax.lax.broadcasted_iota(jnp.int32, (bh, _H, a), 2)
        w0 = jnp.where(lane == 0, 1.0, 0.0).astype(jnp.float32)
        wt_ref[...] = w0
        wtd_ref[...] = w0
        mem_ref[...] = jnp.full_like(mem_ref, 0.01)

    state = state_ref[...]
    wt = wt_ref[...]
    wt_d = wtd_ref[...]
    mem = mem_ref[...]

    hi = jax.lax.Precision.HIGHEST

    read = jax.lax.dot_general(
        wt.astype(jnp.bfloat16), mem.astype(jnp.bfloat16),
        (((2,), (2,)), ((0,), (0,))),
        preferred_element_type=jnp.float32)
    read_flat = read.reshape(bh, _READ)

    x = x_ref[0]

    def bf16x3(lhs, w_hi_ref, w_lo_ref):
        lhs_hi = lhs.astype(jnp.bfloat16)
        lhs_lo = (lhs - lhs_hi.astype(jnp.float32)).astype(jnp.bfloat16)
        w_hi = w_hi_ref[...]
        return (jnp.dot(lhs_hi, w_hi, preferred_element_type=jnp.float32)
                + jnp.dot(lhs_hi, w_lo_ref[...], preferred_element_type=jnp.float32)
                + jnp.dot(lhs_lo, w_hi, preferred_element_type=jnp.float32))

    z = (bf16x3(x, wx_ref, wx_lo_ref)
         + bf16x3(state, wst_ref, wst_lo_ref)
         + bf16x3(read_flat, wr_ref, wr_lo_ref)
         + b_ref[...])

    new_state = jax.nn.sigmoid(z[:, 0:_ST])
    out = z[:, _ST:_ST + _DB]

    s_ = jax.nn.softmax(jax.nn.softplus(z[:, _S_OFF:_S_OFF + 12].reshape(bh, _H, _SH)), axis=-1)
    jd = jax.nn.sigmoid(z[:, _JD_OFF:_JD_OFF + _H])[:, :, None]
    j = jax.nn.softmax(z[:, _J_OFF:_J_OFF + 12].reshape(bh, _H, _SH), axis=-1)
    gamma = (1.0 + jax.nn.softplus(z[:, _GA_OFF:_GA_OFF + _H]))[:, :, None]
    beta = jax.nn.softplus(z[:, _BE_OFF:_BE_OFF + _H])[:, :, None]
    g = jax.nn.sigmoid(z[:, _G_OFF:_G_OFF + _H])[:, :, None]
    erase = jax.nn.sigmoid(z[:, _E_OFF:_E_OFF + _READ]).reshape(bh, _H, _M)
    add = z[:, _AD_OFF:_AD_OFF + _READ].reshape(bh, _H, _M)
    k = jnp.tanh(z[:, _K_OFF:_K_OFF + _READ]).reshape(bh, _H, _M)

    wt_d_new = (1.0 - jd) * wt_d + jd * wt
    lane = jax.lax.broadcasted_iota(jnp.int32, (bh, _H, a), 2)
    wt_addr0 = jnp.where(lane == 0, 1.0, 0.0).astype(jnp.float32)
    wt_j = j[..., 0:1] * wt + j[..., 1:2] * wt_d_new + j[..., 2:3] * wt_addr0

    erase_t = jnp.swapaxes(erase, 1, 2)
    add_t = jnp.swapaxes(add, 1, 2)
    keep = 1.0
    addterm = 0.0
    for h in range(_H):
        wj_h = wt_j[:, h:h + 1, :]
        keep = keep * (1.0 - erase_t[:, :, h:h + 1] * wj_h)
        addterm = addterm + add_t[:, :, h:h + 1] * wj_h
    mem_new = mem * keep + addterm

    kn = k / (jnp.sqrt(jnp.sum(k * k, axis=-1, keepdims=True)) + _EPS)
    mn = mem_new / (jnp.sqrt(jnp.sum(mem_new * mem_new, axis=1, keepdims=True)) + _EPS)
    wt_k = jax.lax.dot_general(
        kn.astype(jnp.bfloat16), mn.astype(jnp.bfloat16),
        (((2,), (1,)), ((0,), (0,))),
        preferred_element_type=jnp.float32)
    wt_b = jax.nn.softmax(beta * wt_k, axis=-1)
    wt_c = g * wt_b + (1.0 - g) * wt_j

    left = jnp.concatenate([wt_c[..., 1:], wt_c[..., :1]], axis=-1)
    right = jnp.concatenate([wt_c[..., -1:], wt_c[..., :-1]], axis=-1)
    wt_s = s_[..., 0:1] * left + s_[..., 1:2] * wt_c + s_[..., 2:3] * right
    wt_sh = jnp.exp(gamma * jnp.log(wt_s + _EPS))
    wt_new = wt_sh / jnp.sum(wt_sh, axis=-1, keepdims=True)

    state_ref[...] = new_state
    wt_ref[...] = wt_new
    wtd_ref[...] = wt_d_new
    mem_ref[...] = mem_new
    out_ref[0] = out


def _run_scan(xs, Wx, Wx_lo, Wst, Wst_lo, Wr, Wr_lo, b_all):
    T, BH, _ = xs.shape
    A = T
    wspec = lambda r: pl.BlockSpec((r, _ZW), lambda t: (0, 0))
    return pl.pallas_call(
        _dwm_step_kernel,
        grid=(T,),
        in_specs=[
            pl.BlockSpec((1, BH, _IN), lambda t: (t, 0, 0)),
            wspec(_IN), wspec(_IN),
            wspec(_ST), wspec(_ST),
            wspec(_READ), wspec(_READ),
            wspec(1),
        ],
        out_specs=pl.BlockSpec((1, BH, _DB), lambda t: (t, 0, 0)),
        out_shape=jax.ShapeDtypeStruct((T, BH, _DB), jnp.float32),
        scratch_shapes=[
            pltpu.VMEM((BH, _ST), jnp.float32),
            pltpu.VMEM((BH, _H, A), jnp.float32),
            pltpu.VMEM((BH, _H, A), jnp.float32),
            pltpu.VMEM((BH, _M, A), jnp.float32),
        ],
        compiler_params=pltpu.CompilerParams(
            dimension_semantics=("arbitrary",),
        ),
    )(xs, Wx, Wx_lo, Wst, Wst_lo, Wr, Wr_lo, b_all)


def kernel(inputs, targets, Ws, bs, Wo, bo, Wu, bu):
    del targets
    B, T, _ = inputs.shape

    W_all, b_all = _pack_weights(Ws, Wo, Wu, bs, bo, bu)
    W_hi = W_all.astype(jnp.bfloat16)
    W_lo = (W_all - W_hi.astype(jnp.float32)).astype(jnp.bfloat16)
    Wx, Wx_lo = W_hi[0:_IN], W_lo[0:_IN]
    Wst, Wst_lo = W_hi[_IN:_IN + _ST], W_lo[_IN:_IN + _ST]
    Wr, Wr_lo = W_hi[_IN + _ST:_COMB], W_lo[_IN + _ST:_COMB]

    xs = jnp.swapaxes(inputs, 0, 1)

    devs = jax.devices()
    if len(devs) >= 2 and B % 2 == 0:
        mesh = jax.sharding.Mesh(devs[:2], ("b",))
        P = jax.sharding.PartitionSpec
        outs = jax.experimental.shard_map.shard_map(
            _run_scan,
            mesh=mesh,
            in_specs=(P(None, "b", None),) + (P(None, None),) * 7,
            out_specs=P(None, "b", None),
            check_rep=False,
        )(xs, Wx, Wx_lo, Wst, Wst_lo, Wr, Wr_lo, b_all)
    else:
        outs = _run_scan(xs, Wx, Wx_lo, Wst, Wst_lo, Wr, Wr_lo, b_all)

    return jnp.swapaxes(outs, 0, 1)


# speedup vs baseline: 3.0308x; 1.0561x over previous
import jax
import jax.numpy as jnp
import jax.experimental.shard_map
from jax.experimental import pallas as pl
from jax.experimental.pallas import tpu as pltpu

_CB, _DB = 2, 256
_IN = _CB + _DB
_ST = 256
_H, _M, _SH = 4, 32, 3
_READ = _H * _M
_COMB = _IN + _ST + _READ
_UPH = 106
_EPS = 1e-12
_ZW = 1024

_S_OFF = 512
_JD_OFF = 524
_J_OFF = 528
_GA_OFF = 540
_BE_OFF = 544
_G_OFF = 548
_E_OFF = 640
_AD_OFF = 768
_K_OFF = 896


def _pack_weights(Ws, Wo, Wu, bs, bo, bu):
    Wu_r = Wu.reshape(_COMB, _H, _UPH)
    bu_r = bu.reshape(_H, _UPH)
    zpad = jnp.zeros((_COMB, 88), jnp.float32)

    def grab(lo, hi):
        return Wu_r[:, :, lo:hi].reshape(_COMB, _H * (hi - lo))

    def tgrab(lo, hi):
        return jnp.swapaxes(Wu_r[:, :, lo:hi], 1, 2).reshape(_COMB, _H * (hi - lo))

    W_all = jnp.concatenate(
        [
            Ws, Wo,
            tgrab(0, 3),
            Wu_r[:, :, 3],
            tgrab(4, 7),
            Wu_r[:, :, 7],
            Wu_r[:, :, 104],
            Wu_r[:, :, 105],
            zpad,
            grab(8, 40),
            grab(40, 72),
            grab(72, 104),
        ],
        axis=1,
    )
    b_all = jnp.concatenate(
        [
            bs, bo,
            bu_r[:, 0:3].T.reshape(-1), bu_r[:, 3], bu_r[:, 4:7].T.reshape(-1),
            bu_r[:, 7], bu_r[:, 104], bu_r[:, 105], jnp.zeros((88,), jnp.float32),
            bu_r[:, 8:40].reshape(-1), bu_r[:, 40:72].reshape(-1),
            bu_r[:, 72:104].reshape(-1),
        ]
    ).reshape(1, _ZW)
    return W_all, b_all


def _dwm_step_kernel(x_ref, wx_ref, wx_lo_ref, wst_ref, wst_lo_ref,
                     wr_ref, wr_lo_ref, b_ref, out_ref,
                     state_ref, wt_ref, wtd_ref, mem_ref):
    t = pl.program_id(0)
    bh = state_ref.shape[0]
    a = wt_ref.shape[2]

    @pl.when(t == 0)
    def _init():
        state_ref[...] = jnp.ones_like(state_ref)
        lane = jax.lax.broadcasted_iota(jnp.int32, (bh, _H, a), 2)
        w0 = jnp.where(lane == 0, 1.0, 0.0).astype(jnp.float32)
        wt_ref[...] = w0
        wtd_ref[...] = w0
        mem_ref[...] = jnp.full_like(mem_ref, 0.01)

    state = state_ref[...]
    wt = wt_ref[...]
    wt_d = wtd_ref[...]
    mem = mem_ref[...]

    hi = jax.lax.Precision.HIGHEST

    read = jax.lax.dot_general(
        wt.astype(jnp.bfloat16), mem.astype(jnp.bfloat16),
        (((2,), (2,)), ((0,), (0,))),
        preferred_element_type=jnp.float32)
    read_flat = read.reshape(bh, _READ)

    x = x_ref[0]

    def bf16x3(lhs, w_hi_ref, w_lo_ref):
        lhs_hi = lhs.astype(jnp.bfloat16)
        lhs_lo = (lhs - lhs_hi.astype(jnp.float32)).astype(jnp.bfloat16)
        w_hi = w_hi_ref[...]
        return (jnp.dot(lhs_hi, w_hi, preferred_element_type=jnp.float32)
                + jnp.dot(lhs_hi, w_lo_ref[...], preferred_element_type=jnp.float32)
                + jnp.dot(lhs_lo, w_hi, preferred_element_type=jnp.float32))

    z = (bf16x3(x, wx_ref, wx_lo_ref)
         + bf16x3(state, wst_ref, wst_lo_ref)
         + bf16x3(read_flat, wr_ref, wr_lo_ref)
         + b_ref[...])

    new_state = jax.nn.sigmoid(z[:, 0:_ST])
    out = z[:, _ST:_ST + _DB]

    def softmax3(c0, c1, c2):
        m = jnp.maximum(c0, jnp.maximum(c1, c2))
        e0 = jnp.exp(c0 - m)
        e1 = jnp.exp(c1 - m)
        e2 = jnp.exp(c2 - m)
        inv = 1.0 / (e0 + e1 + e2)
        return (e0 * inv)[:, :, None], (e1 * inv)[:, :, None], (e2 * inv)[:, :, None]

    s0, s1, s2 = softmax3(
        jax.nn.softplus(z[:, _S_OFF:_S_OFF + _H]),
        jax.nn.softplus(z[:, _S_OFF + _H:_S_OFF + 2 * _H]),
        jax.nn.softplus(z[:, _S_OFF + 2 * _H:_S_OFF + 3 * _H]))
    jd = jax.nn.sigmoid(z[:, _JD_OFF:_JD_OFF + _H])[:, :, None]
    j0, j1, j2 = softmax3(
        z[:, _J_OFF:_J_OFF + _H],
        z[:, _J_OFF + _H:_J_OFF + 2 * _H],
        z[:, _J_OFF + 2 * _H:_J_OFF + 3 * _H])
    gamma = (1.0 + jax.nn.softplus(z[:, _GA_OFF:_GA_OFF + _H]))[:, :, None]
    beta = jax.nn.softplus(z[:, _BE_OFF:_BE_OFF + _H])[:, :, None]
    g = jax.nn.sigmoid(z[:, _G_OFF:_G_OFF + _H])[:, :, None]
    erase = jax.nn.sigmoid(z[:, _E_OFF:_E_OFF + _READ]).reshape(bh, _H, _M)
    add = z[:, _AD_OFF:_AD_OFF + _READ].reshape(bh, _H, _M)
    k = jnp.tanh(z[:, _K_OFF:_K_OFF + _READ]).reshape(bh, _H, _M)

    wt_d_new = (1.0 - jd) * wt_d + jd * wt
    lane = jax.lax.broadcasted_iota(jnp.int32, (bh, _H, a), 2)
    wt_addr0 = jnp.where(lane == 0, 1.0, 0.0).astype(jnp.float32)
    wt_j = j0 * wt + j1 * wt_d_new + j2 * wt_addr0

    erase_t = jnp.swapaxes(erase, 1, 2)
    add_t = jnp.swapaxes(add, 1, 2)
    keep = 1.0
    addterm = 0.0
    for h in range(_H):
        wj_h = wt_j[:, h:h + 1, :]
        keep = keep * (1.0 - erase_t[:, :, h:h + 1] * wj_h)
        addterm = addterm + add_t[:, :, h:h + 1] * wj_h
    mem_new = mem * keep + addterm

    kn = k / (jnp.sqrt(jnp.sum(k * k, axis=-1, keepdims=True)) + _EPS)
    inv_mn = 1.0 / (jnp.sqrt(jnp.sum(mem_new * mem_new, axis=1, keepdims=True)) + _EPS)
    wt_k = jax.lax.dot_general(
        kn.astype(jnp.bfloat16), mem_new.astype(jnp.bfloat16),
        (((2,), (1,)), ((0,), (0,))),
        preferred_element_type=jnp.float32) * inv_mn
    wt_b = jax.nn.softmax(beta * wt_k, axis=-1)
    wt_c = g * wt_b + (1.0 - g) * wt_j

    left = jnp.concatenate([wt_c[..., 1:], wt_c[..., :1]], axis=-1)
    right = jnp.concatenate([wt_c[..., -1:], wt_c[..., :-1]], axis=-1)
    wt_s = s0 * left + s1 * wt_c + s2 * right
    wt_sh = jnp.exp(gamma * jnp.log(wt_s + _EPS))
    wt_new = wt_sh / jnp.sum(wt_sh, axis=-1, keepdims=True)

    state_ref[...] = new_state
    wt_ref[...] = wt_new
    wtd_ref[...] = wt_d_new
    mem_ref[...] = mem_new
    out_ref[0] = out


def _run_scan(xs, Wx, Wx_lo, Wst, Wst_lo, Wr, Wr_lo, b_all):
    T, BH, _ = xs.shape
    A = T
    wspec = lambda r: pl.BlockSpec((r, _ZW), lambda t: (0, 0))
    return pl.pallas_call(
        _dwm_step_kernel,
        grid=(T,),
        in_specs=[
            pl.BlockSpec((1, BH, _IN), lambda t: (t, 0, 0)),
            wspec(_IN), wspec(_IN),
            wspec(_ST), wspec(_ST),
            wspec(_READ), wspec(_READ),
            wspec(1),
        ],
        out_specs=pl.BlockSpec((1, BH, _DB), lambda t: (t, 0, 0)),
        out_shape=jax.ShapeDtypeStruct((T, BH, _DB), jnp.float32),
        scratch_shapes=[
            pltpu.VMEM((BH, _ST), jnp.float32),
            pltpu.VMEM((BH, _H, A), jnp.float32),
            pltpu.VMEM((BH, _H, A), jnp.float32),
            pltpu.VMEM((BH, _M, A), jnp.float32),
        ],
        compiler_params=pltpu.CompilerParams(
            dimension_semantics=("arbitrary",),
        ),
    )(xs, Wx, Wx_lo, Wst, Wst_lo, Wr, Wr_lo, b_all)


def kernel(inputs, targets, Ws, bs, Wo, bo, Wu, bu):
    del targets
    B, T, _ = inputs.shape

    W_all, b_all = _pack_weights(Ws, Wo, Wu, bs, bo, bu)
    W_hi = W_all.astype(jnp.bfloat16)
    W_lo = (W_all - W_hi.astype(jnp.float32)).astype(jnp.bfloat16)
    Wx, Wx_lo = W_hi[0:_IN], W_lo[0:_IN]
    Wst, Wst_lo = W_hi[_IN:_IN + _ST], W_lo[_IN:_IN + _ST]
    Wr, Wr_lo = W_hi[_IN + _ST:_COMB], W_lo[_IN + _ST:_COMB]

    xs = jnp.swapaxes(inputs, 0, 1)

    devs = jax.devices()
    if len(devs) >= 2 and B % 2 == 0:
        mesh = jax.sharding.Mesh(devs[:2], ("b",))
        P = jax.sharding.PartitionSpec
        outs = jax.experimental.shard_map.shard_map(
            _run_scan,
            mesh=mesh,
            in_specs=(P(None, "b", None),) + (P(None, None),) * 7,
            out_specs=P(None, "b", None),
            check_rep=False,
        )(xs, Wx, Wx_lo, Wst, Wst_lo, Wr, Wr_lo, b_all)
    else:
        outs = _run_scan(xs, Wx, Wx_lo, Wst, Wst_lo, Wr, Wr_lo, b_all)

    return jnp.swapaxes(outs, 0, 1)
